# baseline (speedup 1.0000x reference)
import functools

import jax
import jax.numpy as jnp
from jax import lax
from jax.experimental import pallas as pl
from jax.experimental.pallas import tpu as pltpu
from jax.experimental.pallas import tpu_sc as plsc

B, NF, L = 4096, 7, 50
V, D = 1000000, 31
OUT = 3
N_ROWS = B * NF * L
NC, NS = 2, 16
NW = NC * NS
ROWS_PER_W = N_ROWS // NW
CHUNK = 128
N_CHUNKS = ROWS_PER_W // CHUNK


DP = 32


def _sc_gather_body(table_hbm, idx_hbm, out_hbm, idx_v, rows_v, sem):
    wid = lax.axis_index("s") * NC + lax.axis_index("c")
    base = wid * ROWS_PER_W

    def body(c, carry):
        row0 = base + c * CHUNK
        pltpu.sync_copy(idx_hbm.at[pl.ds(row0, CHUNK)], idx_v)
        pltpu.async_copy(table_hbm.at[idx_v], rows_v, sem).wait()
        pltpu.sync_copy(rows_v, out_hbm.at[pl.ds(row0, CHUNK)])
        return carry

    lax.fori_loop(0, N_CHUNKS, body, 0)


@functools.cache
def _sc_gather():
    return pl.kernel(
        _sc_gather_body,
        out_type=jax.ShapeDtypeStruct((N_ROWS, DP), jnp.float32),
        mesh=plsc.VectorSubcoreMesh(core_axis_name="c", subcore_axis_name="s",
                                    num_cores=NC, num_subcores=NS),
        scratch_types=[
            pltpu.VMEM((CHUNK,), jnp.int32),
            pltpu.VMEM((CHUNK, DP), jnp.float32),
            pltpu.SemaphoreType.DMA,
        ],
        compiler_params=pltpu.CompilerParams(use_tc_tiling_on_sc=False),
    )


def _attn_encode_body(emb_ref, w_ref, encw_ref, encb_ref, out_ref):
    i = pl.program_id(0)
    xt = jnp.tanh(emb_ref[0][:, :D])
    e = jnp.sum(xt * w_ref[0, 0], axis=1, keepdims=True)
    p = jnp.exp(e - jnp.max(e))
    a = p * (1.0 / jnp.sum(p))
    contrib = jnp.dot(xt * a, encw_ref[0],
                      preferred_element_type=jnp.float32)
    prev = jnp.where(i == 0, 0.0, out_ref[...])
    tot = prev + contrib
    is_last = i == NF * L - 1
    out_ref[...] = jnp.where(is_last,
                             jnp.maximum(tot + encb_ref[...], 0.0), tot)


def kernel(x, tables, attn_w, attn_b, enc_w, enc_b):
    del attn_b
    table_flat = jnp.pad(tables.reshape(NF * V, D), ((0, 0), (0, DP - D)))
    idx = (jnp.transpose(x, (1, 2, 0))
           + (jnp.arange(NF, dtype=jnp.int32) * V)[:, None, None])
    idx_flat = idx.reshape(N_ROWS)

    emb = _sc_gather()(table_flat, idx_flat)
    emb3 = emb.reshape(NF * L, B, DP)

    out = pl.pallas_call(
        _attn_encode_body,
        grid=(NF * L,),
        in_specs=[
            pl.BlockSpec((1, B, DP), lambda i: (i, 0, 0)),
            pl.BlockSpec((1, 1, D), lambda i: (i // L, 0, 0)),
            pl.BlockSpec((1, D, OUT), lambda i: (i // L, 0, 0)),
            pl.BlockSpec((1, OUT), lambda i: (0, 0)),
        ],
        out_specs=pl.BlockSpec((B, OUT), lambda i: (0, 0)),
        out_shape=jax.ShapeDtypeStruct((B, OUT), jnp.float32),
    )(emb3, attn_w.reshape(NF, 1, D), enc_w.reshape(NF, D, OUT),
      enc_b.reshape(1, OUT))
    return out
